# baseline (speedup 1.0000x reference)
import functools

import jax
import jax.numpy as jnp
from jax import lax
from jax.experimental import pallas as pl
from jax.experimental.pallas import tpu as pltpu
from jax.experimental.pallas import tpu_sc as plsc

RES = 32
NVOX = RES ** 3
B, C, N = 4, 64, 65536
CH = 2048


def _prep_body(x1c_ref, x2c_ref, idxp_ref, frac_ref):
    r = float(RES)

    def norm_coords(c):
        mean = jnp.mean(c, axis=1, keepdims=True)
        nc = c - mean
        norm = jnp.sqrt(jnp.sum(nc * nc, axis=0, keepdims=True))
        denom = jnp.max(norm) * 2.0
        nc = nc / denom + 0.5
        return jnp.clip(nc * r, 0.0, r - 1.0)

    nc1 = norm_coords(x1c_ref[0])
    nc2 = norm_coords(x2c_ref[0])

    vox2 = jnp.round(nc2).astype(jnp.int32)
    idx2 = vox2[0:1] * (RES * RES) + vox2[1:2] * RES + vox2[2:3]

    lo = jnp.floor(nc1)
    frac_ref[0] = nc1 - lo
    lo_i = lo.astype(jnp.int32)
    idx000 = lo_i[0:1] * (RES * RES) + lo_i[1:2] * RES + lo_i[2:3]
    idxp_ref[0] = jnp.concatenate([idx000, idx2], axis=0)


def _prep(x1_coords, x2_coords):
    return pl.pallas_call(
        _prep_body,
        grid=(B,),
        in_specs=[
            pl.BlockSpec((1, 3, N), lambda b: (b, 0, 0)),
            pl.BlockSpec((1, 3, N), lambda b: (b, 0, 0)),
        ],
        out_specs=[
            pl.BlockSpec((1, 2, N), lambda b: (b, 0, 0)),
            pl.BlockSpec((1, 3, N), lambda b: (b, 0, 0)),
        ],
        out_shape=[
            jax.ShapeDtypeStruct((B, 2, N), jnp.int32),
            jax.ShapeDtypeStruct((B, 3, N), jnp.float32),
        ],
    )(x1_coords, x2_coords)


def _sc_body(x1f, x2f, idxp, frac, out,
             sum0, sum1, cnt,
             ib0, ib1, fa0, fa1, fb0, fb1, fc0, fc1,
             oa0, oa1, ob0, ob1, sin0, sin1, sout0, sout1, sx1):
    wid = lax.axis_index("s") * 2 + lax.axis_index("c")
    c0 = wid * 2
    c1 = c0 + 1
    zeros16 = jnp.zeros((16,), jnp.float32)
    ones16 = jnp.ones((16,), jnp.float32)
    NCH = N // CH

    ib = (ib0, ib1)
    fa = (fa0, fa1)
    fb = (fb0, fb1)
    fc = (fc0, fc1)
    oa = (oa0, oa1)
    ob = (ob0, ob1)
    sin = (sin0, sin1)
    sout = (sout0, sout1)

    def batch_body(b, _):
        pltpu.async_copy(x1f.at[pl.ds((b * C + c0) * N, N)],
                         out.at[pl.ds((b * 2 * C + c0) * N, N)], sx1)
        pltpu.async_copy(x1f.at[pl.ds((b * C + c1) * N, N)],
                         out.at[pl.ds((b * 2 * C + c1) * N, N)], sx1)

        def zero_body(i, _):
            for u in range(4):
                ds = pl.ds(i * 64 + u * 16, 16)
                sum0[ds] = zeros16
                sum1[ds] = zeros16
                cnt[ds] = zeros16
            return ()
        lax.fori_loop(0, NVOX // 64, zero_body, ())

        def scat_in(ci, par):
            n0 = ci * CH
            pltpu.async_copy(idxp.at[pl.ds((b * 2 + 1) * N + n0, CH)],
                             ib[par], sin[par])
            pltpu.async_copy(x2f.at[pl.ds((b * C + c0) * N + n0, CH)],
                             fa[par], sin[par])
            pltpu.async_copy(x2f.at[pl.ds((b * C + c1) * N + n0, CH)],
                             fb[par], sin[par])

        def scat_wait(par):
            src = idxp.at[pl.ds(0, CH)]
            pltpu.make_async_copy(src, ib[par], sin[par]).wait()
            srcf = x2f.at[pl.ds(0, CH)]
            pltpu.make_async_copy(srcf, fa[par], sin[par]).wait()
            pltpu.make_async_copy(srcf, fb[par], sin[par]).wait()

        scat_in(0, 0)
        scat_in(1, 1)

        def scat_chunk(ci0, _):
            for par in (0, 1):
                ci = ci0 * 2 + par
                scat_wait(par)

                def scat_g(g, _):
                    for u in range(2):
                        ds = pl.ds(g * 32 + u * 16, 16)
                        iv = ib[par][ds]
                        plsc.addupdate_scatter(sum0, [iv], fa[par][ds])
                        plsc.addupdate_scatter(sum1, [iv], fb[par][ds])
                        plsc.addupdate_scatter(cnt, [iv], ones16)
                    return ()
                lax.fori_loop(0, CH // 32, scat_g, ())

                @pl.when(ci + 2 < NCH)
                def _():
                    scat_in(ci + 2, par)
            return ()
        lax.fori_loop(0, NCH // 2, scat_chunk, ())

        def div_body(i, _):
            for u in range(2):
                ds = pl.ds(i * 32 + u * 16, 16)
                c = jnp.maximum(cnt[ds], 1.0)
                sum0[ds] = sum0[ds] / c
                sum1[ds] = sum1[ds] / c
            return ()
        lax.fori_loop(0, NVOX // 32, div_body, ())

        def gath_in(ci, par):
            n0 = ci * CH
            pltpu.async_copy(idxp.at[pl.ds(b * 2 * N + n0, CH)],
                             ib[par], sin[par])
            pltpu.async_copy(frac.at[pl.ds(b * 3 * N + n0, CH)],
                             fa[par], sin[par])
            pltpu.async_copy(frac.at[pl.ds((b * 3 + 1) * N + n0, CH)],
                             fb[par], sin[par])
            pltpu.async_copy(frac.at[pl.ds((b * 3 + 2) * N + n0, CH)],
                             fc[par], sin[par])

        def gath_wait(par):
            src = idxp.at[pl.ds(0, CH)]
            pltpu.make_async_copy(src, ib[par], sin[par]).wait()
            srcf = frac.at[pl.ds(0, CH)]
            pltpu.make_async_copy(srcf, fa[par], sin[par]).wait()
            pltpu.make_async_copy(srcf, fb[par], sin[par]).wait()
            pltpu.make_async_copy(srcf, fc[par], sin[par]).wait()

        def out_issue(ci, par):
            n0 = ci * CH
            pltpu.async_copy(
                oa[par], out.at[pl.ds((b * 2 * C + C + c0) * N + n0, CH)],
                sout[par])
            pltpu.async_copy(
                ob[par], out.at[pl.ds((b * 2 * C + C + c1) * N + n0, CH)],
                sout[par])

        def out_wait(par):
            dst = out.at[pl.ds(0, CH)]
            pltpu.make_async_copy(oa[par], dst, sout[par]).wait()
            pltpu.make_async_copy(ob[par], dst, sout[par]).wait()

        gath_in(0, 0)
        gath_in(1, 1)

        def gath_chunk(ci0, _):
            for par in (0, 1):
                ci = ci0 * 2 + par
                gath_wait(par)

                @pl.when(ci >= 2)
                def _():
                    out_wait(par)

                def gath_g(g, _):
                    for u in range(2):
                        ds = pl.ds(g * 32 + u * 16, 16)
                        i000 = ib[par][ds]
                        vfx = fa[par][ds]
                        vfy = fb[par][ds]
                        vfz = fc[par][ds]
                        dx = jnp.where(i000 < (RES - 1) * RES * RES,
                                       RES * RES, 0)
                        dy = jnp.where((i000 & (RES * RES - 1)) < (RES - 1) * RES,
                                       RES, 0)
                        dz = jnp.where((i000 & (RES - 1)) < RES - 1, 1, 0)
                        gx = 1.0 - vfx
                        gy = 1.0 - vfy
                        gz = 1.0 - vfz
                        i0 = i000
                        i1 = i000 + dy
                        i2 = i000 + dx
                        i3 = i2 + dy
                        acc0 = zeros16
                        acc1 = zeros16
                        for ibase, wxy in ((i0, gx * gy), (i1, gx * vfy),
                                           (i2, vfx * gy), (i3, vfx * vfy)):
                            wlo = wxy * gz
                            whi = wxy * vfz
                            ihi = ibase + dz
                            acc0 = acc0 + wlo * plsc.load_gather(sum0, [ibase])
                            acc0 = acc0 + whi * plsc.load_gather(sum0, [ihi])
                            acc1 = acc1 + wlo * plsc.load_gather(sum1, [ibase])
                            acc1 = acc1 + whi * plsc.load_gather(sum1, [ihi])
                        oa[par][ds] = acc0
                        ob[par][ds] = acc1
                    return ()
                lax.fori_loop(0, CH // 32, gath_g, ())

                @pl.when(ci + 2 < NCH)
                def _():
                    gath_in(ci + 2, par)

                out_issue(ci, par)
            return ()
        lax.fori_loop(0, NCH // 2, gath_chunk, ())
        out_wait(0)
        out_wait(1)
        src1 = x1f.at[pl.ds(0, N)]
        dst1 = out.at[pl.ds(0, N)]
        pltpu.make_async_copy(src1, dst1, sx1).wait()
        pltpu.make_async_copy(src1, dst1, sx1).wait()
        return ()

    lax.fori_loop(0, B, batch_body, ())


def _devoxelize(x1_features, x2_features, idxp, frac):
    mesh = plsc.VectorSubcoreMesh(core_axis_name="c", subcore_axis_name="s")
    f = pl.kernel(
        _sc_body,
        mesh=mesh,
        compiler_params=pltpu.CompilerParams(needs_layout_passes=False),
        out_type=jax.ShapeDtypeStruct((B * 2 * C * N,), jnp.float32),
        scratch_types=[
            pltpu.VMEM((NVOX,), jnp.float32),
            pltpu.VMEM((NVOX,), jnp.float32),
            pltpu.VMEM((NVOX,), jnp.float32),
            pltpu.VMEM((CH,), jnp.int32),
            pltpu.VMEM((CH,), jnp.int32),
            pltpu.VMEM((CH,), jnp.float32),
            pltpu.VMEM((CH,), jnp.float32),
            pltpu.VMEM((CH,), jnp.float32),
            pltpu.VMEM((CH,), jnp.float32),
            pltpu.VMEM((CH,), jnp.float32),
            pltpu.VMEM((CH,), jnp.float32),
            pltpu.VMEM((CH,), jnp.float32),
            pltpu.VMEM((CH,), jnp.float32),
            pltpu.VMEM((CH,), jnp.float32),
            pltpu.VMEM((CH,), jnp.float32),
            pltpu.SemaphoreType.DMA,
            pltpu.SemaphoreType.DMA,
            pltpu.SemaphoreType.DMA,
            pltpu.SemaphoreType.DMA,
            pltpu.SemaphoreType.DMA,
        ],
    )
    flat = f(x1_features.reshape(-1), x2_features.reshape(-1),
             idxp.reshape(-1), frac.reshape(-1))
    return flat.reshape(B, 2 * C, N)


def kernel(x1_features, x2_features, x1_coords, x2_coords):
    idxp, frac = _prep(x1_coords, x2_coords)
    return _devoxelize(x1_features, x2_features, idxp, frac)


# speedup vs baseline: 2.6319x; 2.6319x over previous
import functools

import jax
import jax.numpy as jnp
from jax import lax
from jax.experimental import pallas as pl
from jax.experimental.pallas import tpu as pltpu
from jax.experimental.pallas import tpu_sc as plsc

RES = 32
NVOX = RES ** 3
B, C, N = 4, 64, 65536
CH = 2048


def _prep_body(x1c_ref, x2c_ref, idxp_ref, frac_ref):
    r = float(RES)

    def norm_coords(c):
        mean = jnp.mean(c, axis=1, keepdims=True)
        nc = c - mean
        norm = jnp.sqrt(jnp.sum(nc * nc, axis=0, keepdims=True))
        denom = jnp.max(norm) * 2.0
        nc = nc / denom + 0.5
        return jnp.clip(nc * r, 0.0, r - 1.0)

    nc1 = norm_coords(x1c_ref[0])
    nc2 = norm_coords(x2c_ref[0])

    vox2 = jnp.round(nc2).astype(jnp.int32)
    idx2 = vox2[0:1] * (RES * RES) + vox2[1:2] * RES + vox2[2:3]

    lo = jnp.floor(nc1)
    frac_ref[0] = nc1 - lo
    lo_i = lo.astype(jnp.int32)
    idx000 = lo_i[0:1] * (RES * RES) + lo_i[1:2] * RES + lo_i[2:3]
    idxp_ref[0] = jnp.concatenate([idx000, idx2], axis=0)


def _prep(x1_coords, x2_coords):
    return pl.pallas_call(
        _prep_body,
        grid=(B,),
        in_specs=[
            pl.BlockSpec((1, 3, N), lambda b: (b, 0, 0)),
            pl.BlockSpec((1, 3, N), lambda b: (b, 0, 0)),
        ],
        out_specs=[
            pl.BlockSpec((1, 2, N), lambda b: (b, 0, 0)),
            pl.BlockSpec((1, 3, N), lambda b: (b, 0, 0)),
        ],
        out_shape=[
            jax.ShapeDtypeStruct((B, 2, N), jnp.int32),
            jax.ShapeDtypeStruct((B, 3, N), jnp.float32),
        ],
    )(x1_coords, x2_coords)


def _sc_body(x2f, idxp, frac, out,
             sum0, sum1, cnt,
             ib0, ib1, fa0, fa1, fb0, fb1, fc0, fc1,
             oa0, oa1, ob0, ob1, sin0, sin1, sout0, sout1):
    wid = lax.axis_index("s") * 2 + lax.axis_index("c")
    c0 = wid * 2
    c1 = c0 + 1
    zeros16 = jnp.zeros((16,), jnp.float32)
    ones16 = jnp.ones((16,), jnp.float32)
    NCH = N // CH

    ib = (ib0, ib1)
    fa = (fa0, fa1)
    fb = (fb0, fb1)
    fc = (fc0, fc1)
    oa = (oa0, oa1)
    ob = (ob0, ob1)
    sin = (sin0, sin1)
    sout = (sout0, sout1)

    def batch_body(b, _):
        @plsc.parallel_loop(0, NVOX // 16, unroll=4)
        def _(i):
            ds = pl.ds(i * 16, 16)
            sum0[ds] = zeros16
            sum1[ds] = zeros16
            cnt[ds] = zeros16

        def scat_in(ci, par):
            n0 = ci * CH
            pltpu.async_copy(idxp.at[pl.ds((b * 2 + 1) * N + n0, CH)],
                             ib[par], sin[par])
            pltpu.async_copy(x2f.at[pl.ds((b * C + c0) * N + n0, CH)],
                             fa[par], sin[par])
            pltpu.async_copy(x2f.at[pl.ds((b * C + c1) * N + n0, CH)],
                             fb[par], sin[par])

        def scat_wait(par):
            src = idxp.at[pl.ds(0, CH)]
            pltpu.make_async_copy(src, ib[par], sin[par]).wait()
            srcf = x2f.at[pl.ds(0, CH)]
            pltpu.make_async_copy(srcf, fa[par], sin[par]).wait()
            pltpu.make_async_copy(srcf, fb[par], sin[par]).wait()

        scat_in(0, 0)
        scat_in(1, 1)

        def scat_chunk(ci0, _):
            for par in (0, 1):
                ci = ci0 * 2 + par
                scat_wait(par)

                @plsc.parallel_loop(0, CH // 16, unroll=4)
                def _(g):
                    ds = pl.ds(g * 16, 16)
                    iv = ib[par][ds]
                    plsc.addupdate_scatter(sum0, [iv], fa[par][ds])
                    plsc.addupdate_scatter(sum1, [iv], fb[par][ds])
                    plsc.addupdate_scatter(cnt, [iv], ones16)

                @pl.when(ci + 2 < NCH)
                def _():
                    scat_in(ci + 2, par)
            return ()
        lax.fori_loop(0, NCH // 2, scat_chunk, ())

        @plsc.parallel_loop(0, NVOX // 16, unroll=4)
        def _(i):
            ds = pl.ds(i * 16, 16)
            c = jnp.maximum(cnt[ds], 1.0)
            sum0[ds] = sum0[ds] / c
            sum1[ds] = sum1[ds] / c

        def gath_in(ci, par):
            n0 = ci * CH
            pltpu.async_copy(idxp.at[pl.ds(b * 2 * N + n0, CH)],
                             ib[par], sin[par])
            pltpu.async_copy(frac.at[pl.ds(b * 3 * N + n0, CH)],
                             fa[par], sin[par])
            pltpu.async_copy(frac.at[pl.ds((b * 3 + 1) * N + n0, CH)],
                             fb[par], sin[par])
            pltpu.async_copy(frac.at[pl.ds((b * 3 + 2) * N + n0, CH)],
                             fc[par], sin[par])

        def gath_wait(par):
            src = idxp.at[pl.ds(0, CH)]
            pltpu.make_async_copy(src, ib[par], sin[par]).wait()
            srcf = frac.at[pl.ds(0, CH)]
            pltpu.make_async_copy(srcf, fa[par], sin[par]).wait()
            pltpu.make_async_copy(srcf, fb[par], sin[par]).wait()
            pltpu.make_async_copy(srcf, fc[par], sin[par]).wait()

        def out_issue(ci, par):
            n0 = ci * CH
            pltpu.async_copy(oa[par], out.at[pl.ds((b * C + c0) * N + n0, CH)],
                             sout[par])
            pltpu.async_copy(ob[par], out.at[pl.ds((b * C + c1) * N + n0, CH)],
                             sout[par])

        def out_wait(par):
            dst = out.at[pl.ds(0, CH)]
            pltpu.make_async_copy(oa[par], dst, sout[par]).wait()
            pltpu.make_async_copy(ob[par], dst, sout[par]).wait()

        gath_in(0, 0)
        gath_in(1, 1)

        def gath_chunk(ci0, _):
            for par in (0, 1):
                ci = ci0 * 2 + par
                gath_wait(par)

                @pl.when(ci >= 2)
                def _():
                    out_wait(par)

                @plsc.parallel_loop(0, CH // 16, unroll=2)
                def _(g):
                    ds = pl.ds(g * 16, 16)
                    i000 = ib[par][ds]
                    vfx = fa[par][ds]
                    vfy = fb[par][ds]
                    vfz = fc[par][ds]
                    dx = jnp.where(i000 < (RES - 1) * RES * RES,
                                   RES * RES, 0)
                    dy = jnp.where((i000 & (RES * RES - 1)) < (RES - 1) * RES,
                                   RES, 0)
                    dz = jnp.where((i000 & (RES - 1)) < RES - 1, 1, 0)
                    gx = 1.0 - vfx
                    gy = 1.0 - vfy
                    gz = 1.0 - vfz
                    i0 = i000
                    i1 = i000 + dy
                    i2 = i000 + dx
                    i3 = i2 + dy
                    acc0 = zeros16
                    acc1 = zeros16
                    for ibase, wxy in ((i0, gx * gy), (i1, gx * vfy),
                                       (i2, vfx * gy), (i3, vfx * vfy)):
                        wlo = wxy * gz
                        whi = wxy * vfz
                        ihi = ibase + dz
                        acc0 = acc0 + wlo * plsc.load_gather(sum0, [ibase])
                        acc0 = acc0 + whi * plsc.load_gather(sum0, [ihi])
                        acc1 = acc1 + wlo * plsc.load_gather(sum1, [ibase])
                        acc1 = acc1 + whi * plsc.load_gather(sum1, [ihi])
                    oa[par][ds] = acc0
                    ob[par][ds] = acc1

                @pl.when(ci + 2 < NCH)
                def _():
                    gath_in(ci + 2, par)

                out_issue(ci, par)
            return ()
        lax.fori_loop(0, NCH // 2, gath_chunk, ())
        out_wait(0)
        out_wait(1)
        return ()

    lax.fori_loop(0, B, batch_body, ())


def _devoxelize(x2_features, idxp, frac):
    mesh = plsc.VectorSubcoreMesh(core_axis_name="c", subcore_axis_name="s")
    f = pl.kernel(
        _sc_body,
        mesh=mesh,
        compiler_params=pltpu.CompilerParams(needs_layout_passes=False),
        out_type=jax.ShapeDtypeStruct((B * C * N,), jnp.float32),
        scratch_types=[
            pltpu.VMEM((NVOX,), jnp.float32),
            pltpu.VMEM((NVOX,), jnp.float32),
            pltpu.VMEM((NVOX,), jnp.float32),
            pltpu.VMEM((CH,), jnp.int32),
            pltpu.VMEM((CH,), jnp.int32),
            pltpu.VMEM((CH,), jnp.float32),
            pltpu.VMEM((CH,), jnp.float32),
            pltpu.VMEM((CH,), jnp.float32),
            pltpu.VMEM((CH,), jnp.float32),
            pltpu.VMEM((CH,), jnp.float32),
            pltpu.VMEM((CH,), jnp.float32),
            pltpu.VMEM((CH,), jnp.float32),
            pltpu.VMEM((CH,), jnp.float32),
            pltpu.VMEM((CH,), jnp.float32),
            pltpu.VMEM((CH,), jnp.float32),
            pltpu.SemaphoreType.DMA,
            pltpu.SemaphoreType.DMA,
            pltpu.SemaphoreType.DMA,
            pltpu.SemaphoreType.DMA,
        ],
    )
    flat = f(x2_features.reshape(-1), idxp.reshape(-1), frac.reshape(-1))
    return flat.reshape(B, C, N)


def kernel(x1_features, x2_features, x1_coords, x2_coords):
    idxp, frac = _prep(x1_coords, x2_coords)
    devox = _devoxelize(x2_features, idxp, frac)
    return jnp.concatenate([x1_features, devox], axis=1)


# speedup vs baseline: 2.6883x; 1.0214x over previous
import functools

import jax
import jax.numpy as jnp
from jax import lax
from jax.experimental import pallas as pl
from jax.experimental.pallas import tpu as pltpu
from jax.experimental.pallas import tpu_sc as plsc

RES = 32
NVOX = RES ** 3
B, C, N = 4, 64, 65536
CH = 2048


def _prep_body(x1c_ref, x2c_ref, idxp_ref, frac_ref):
    r = float(RES)

    def norm_coords(c):
        mean = jnp.mean(c, axis=1, keepdims=True)
        nc = c - mean
        norm = jnp.sqrt(jnp.sum(nc * nc, axis=0, keepdims=True))
        denom = jnp.max(norm) * 2.0
        nc = nc / denom + 0.5
        return jnp.clip(nc * r, 0.0, r - 1.0)

    nc1 = norm_coords(x1c_ref[0])
    nc2 = norm_coords(x2c_ref[0])

    vox2 = jnp.round(nc2).astype(jnp.int32)
    idx2 = vox2[0:1] * (RES * RES) + vox2[1:2] * RES + vox2[2:3]

    lo = jnp.floor(nc1)
    frac_ref[0] = nc1 - lo
    lo_i = lo.astype(jnp.int32)
    idx000 = lo_i[0:1] * (RES * RES) + lo_i[1:2] * RES + lo_i[2:3]
    idxp_ref[0] = jnp.concatenate([idx000, idx2], axis=0)


def _prep(x1_coords, x2_coords):
    return pl.pallas_call(
        _prep_body,
        grid=(B,),
        in_specs=[
            pl.BlockSpec((1, 3, N), lambda b: (b, 0, 0)),
            pl.BlockSpec((1, 3, N), lambda b: (b, 0, 0)),
        ],
        out_specs=[
            pl.BlockSpec((1, 2, N), lambda b: (b, 0, 0)),
            pl.BlockSpec((1, 3, N), lambda b: (b, 0, 0)),
        ],
        out_shape=[
            jax.ShapeDtypeStruct((B, 2, N), jnp.int32),
            jax.ShapeDtypeStruct((B, 3, N), jnp.float32),
        ],
    )(x1_coords, x2_coords)


def _sc_body(x2f, idxp, frac, out,
             sum0, sum1, cnt,
             ib0, ib1, fa0, fa1, fb0, fb1, fc0, fc1,
             oa0, oa1, ob0, ob1, sin0, sin1, sout0, sout1):
    wid = lax.axis_index("s") * 2 + lax.axis_index("c")
    c0 = wid * 2
    c1 = c0 + 1
    zeros16 = jnp.zeros((16,), jnp.float32)
    ones16 = jnp.ones((16,), jnp.float32)
    NCH = N // CH

    ib = (ib0, ib1)
    fa = (fa0, fa1)
    fb = (fb0, fb1)
    fc = (fc0, fc1)
    oa = (oa0, oa1)
    ob = (ob0, ob1)
    sin = (sin0, sin1)
    sout = (sout0, sout1)

    def batch_body(b, _):
        @plsc.parallel_loop(0, NVOX // 16, unroll=4)
        def _(i):
            ds = pl.ds(i * 16, 16)
            sum0[ds] = zeros16
            sum1[ds] = zeros16
            cnt[ds] = zeros16

        def scat_in(ci, par):
            n0 = ci * CH
            pltpu.async_copy(idxp.at[pl.ds((b * 2 + 1) * N + n0, CH)],
                             ib[par], sin[par])
            pltpu.async_copy(x2f.at[pl.ds((b * C + c0) * N + n0, CH)],
                             fa[par], sin[par])
            pltpu.async_copy(x2f.at[pl.ds((b * C + c1) * N + n0, CH)],
                             fb[par], sin[par])

        def scat_wait(par):
            src = idxp.at[pl.ds(0, CH)]
            pltpu.make_async_copy(src, ib[par], sin[par]).wait()
            srcf = x2f.at[pl.ds(0, CH)]
            pltpu.make_async_copy(srcf, fa[par], sin[par]).wait()
            pltpu.make_async_copy(srcf, fb[par], sin[par]).wait()

        scat_in(0, 0)
        scat_in(1, 1)

        def scat_chunk(ci0, _):
            for par in (0, 1):
                ci = ci0 * 2 + par
                scat_wait(par)

                @plsc.parallel_loop(0, CH // 16, unroll=8)
                def _(g):
                    ds = pl.ds(g * 16, 16)
                    iv = ib[par][ds]
                    plsc.addupdate_scatter(sum0, [iv], fa[par][ds])
                    plsc.addupdate_scatter(sum1, [iv], fb[par][ds])
                    plsc.addupdate_scatter(cnt, [iv], ones16)

                @pl.when(ci + 2 < NCH)
                def _():
                    scat_in(ci + 2, par)
            return ()
        lax.fori_loop(0, NCH // 2, scat_chunk, ())

        @plsc.parallel_loop(0, NVOX // 16, unroll=4)
        def _(i):
            ds = pl.ds(i * 16, 16)
            c = jnp.maximum(cnt[ds], 1.0)
            sum0[ds] = sum0[ds] / c
            sum1[ds] = sum1[ds] / c

        def gath_in(ci, par):
            n0 = ci * CH
            pltpu.async_copy(idxp.at[pl.ds(b * 2 * N + n0, CH)],
                             ib[par], sin[par])
            pltpu.async_copy(frac.at[pl.ds(b * 3 * N + n0, CH)],
                             fa[par], sin[par])
            pltpu.async_copy(frac.at[pl.ds((b * 3 + 1) * N + n0, CH)],
                             fb[par], sin[par])
            pltpu.async_copy(frac.at[pl.ds((b * 3 + 2) * N + n0, CH)],
                             fc[par], sin[par])

        def gath_wait(par):
            src = idxp.at[pl.ds(0, CH)]
            pltpu.make_async_copy(src, ib[par], sin[par]).wait()
            srcf = frac.at[pl.ds(0, CH)]
            pltpu.make_async_copy(srcf, fa[par], sin[par]).wait()
            pltpu.make_async_copy(srcf, fb[par], sin[par]).wait()
            pltpu.make_async_copy(srcf, fc[par], sin[par]).wait()

        def out_issue(ci, par):
            n0 = ci * CH
            pltpu.async_copy(oa[par], out.at[pl.ds((b * C + c0) * N + n0, CH)],
                             sout[par])
            pltpu.async_copy(ob[par], out.at[pl.ds((b * C + c1) * N + n0, CH)],
                             sout[par])

        def out_wait(par):
            dst = out.at[pl.ds(0, CH)]
            pltpu.make_async_copy(oa[par], dst, sout[par]).wait()
            pltpu.make_async_copy(ob[par], dst, sout[par]).wait()

        gath_in(0, 0)
        gath_in(1, 1)

        def gath_chunk(ci0, _):
            for par in (0, 1):
                ci = ci0 * 2 + par
                gath_wait(par)

                @pl.when(ci >= 2)
                def _():
                    out_wait(par)

                @plsc.parallel_loop(0, CH // 16, unroll=4)
                def _(g):
                    ds = pl.ds(g * 16, 16)
                    i000 = ib[par][ds]
                    vfx = fa[par][ds]
                    vfy = fb[par][ds]
                    vfz = fc[par][ds]
                    dx = jnp.where(i000 < (RES - 1) * RES * RES,
                                   RES * RES, 0)
                    dy = jnp.where((i000 & (RES * RES - 1)) < (RES - 1) * RES,
                                   RES, 0)
                    dz = jnp.where((i000 & (RES - 1)) < RES - 1, 1, 0)
                    gx = 1.0 - vfx
                    gy = 1.0 - vfy
                    gz = 1.0 - vfz
                    i0 = i000
                    i1 = i000 + dy
                    i2 = i000 + dx
                    i3 = i2 + dy
                    acc0 = zeros16
                    acc1 = zeros16
                    for ibase, wxy in ((i0, gx * gy), (i1, gx * vfy),
                                       (i2, vfx * gy), (i3, vfx * vfy)):
                        wlo = wxy * gz
                        whi = wxy * vfz
                        ihi = ibase + dz
                        acc0 = acc0 + wlo * plsc.load_gather(sum0, [ibase])
                        acc0 = acc0 + whi * plsc.load_gather(sum0, [ihi])
                        acc1 = acc1 + wlo * plsc.load_gather(sum1, [ibase])
                        acc1 = acc1 + whi * plsc.load_gather(sum1, [ihi])
                    oa[par][ds] = acc0
                    ob[par][ds] = acc1

                @pl.when(ci + 2 < NCH)
                def _():
                    gath_in(ci + 2, par)

                out_issue(ci, par)
            return ()
        lax.fori_loop(0, NCH // 2, gath_chunk, ())
        out_wait(0)
        out_wait(1)
        return ()

    lax.fori_loop(0, B, batch_body, ())


def _devoxelize(x2_features, idxp, frac):
    mesh = plsc.VectorSubcoreMesh(core_axis_name="c", subcore_axis_name="s")
    f = pl.kernel(
        _sc_body,
        mesh=mesh,
        compiler_params=pltpu.CompilerParams(needs_layout_passes=False),
        out_type=jax.ShapeDtypeStruct((B * C * N,), jnp.float32),
        scratch_types=[
            pltpu.VMEM((NVOX,), jnp.float32),
            pltpu.VMEM((NVOX,), jnp.float32),
            pltpu.VMEM((NVOX,), jnp.float32),
            pltpu.VMEM((CH,), jnp.int32),
            pltpu.VMEM((CH,), jnp.int32),
            pltpu.VMEM((CH,), jnp.float32),
            pltpu.VMEM((CH,), jnp.float32),
            pltpu.VMEM((CH,), jnp.float32),
            pltpu.VMEM((CH,), jnp.float32),
            pltpu.VMEM((CH,), jnp.float32),
            pltpu.VMEM((CH,), jnp.float32),
            pltpu.VMEM((CH,), jnp.float32),
            pltpu.VMEM((CH,), jnp.float32),
            pltpu.VMEM((CH,), jnp.float32),
            pltpu.VMEM((CH,), jnp.float32),
            pltpu.SemaphoreType.DMA,
            pltpu.SemaphoreType.DMA,
            pltpu.SemaphoreType.DMA,
            pltpu.SemaphoreType.DMA,
        ],
    )
    flat = f(x2_features.reshape(-1), idxp.reshape(-1), frac.reshape(-1))
    return flat.reshape(B, C, N)


def kernel(x1_features, x2_features, x1_coords, x2_coords):
    idxp, frac = _prep(x1_coords, x2_coords)
    devox = _devoxelize(x2_features, idxp, frac)
    return jnp.concatenate([x1_features, devox], axis=1)
